# baseline (speedup 1.0000x reference)
import functools

import jax
import jax.numpy as jnp
from jax import lax
from jax.experimental import pallas as pl
from jax.experimental.pallas import tpu as pltpu
from jax.experimental.pallas import tpu_sc as plsc

S = 2048
D = 2048
E = 8
TR = 256
NT = 24
P = NT * TR
TO = 1024
SC = 128
WB = 128

NEG = -jnp.inf


def _router_body(x_ref, wgt_ref, bg_ref, w1_ref, w2_ref, pos1_ref, pos2_ref,
                 te_ref, used_ref):
    xf = x_ref[...]
    logits = lax.dot_general(
        xf, wgt_ref[...], (((1,), (0,)), ((), ())),
        preferred_element_type=jnp.float32,
    ) + bg_ref[...]
    lane = lax.broadcasted_iota(jnp.int32, (S, E), 1)
    m1 = jnp.max(logits, axis=1, keepdims=True)
    i1 = jnp.min(jnp.where(logits == m1, lane, E), axis=1, keepdims=True)
    oh1 = lane == i1
    masked = jnp.where(oh1, NEG, logits)
    m2 = jnp.max(masked, axis=1, keepdims=True)
    i2 = jnp.min(jnp.where(masked == m2, lane, E), axis=1, keepdims=True)
    oh2 = lane == i2
    z = jnp.exp(m2 - m1)
    ones16 = jnp.ones((1, WB), jnp.float32)
    w1_ref[...] = (1.0 / (1.0 + z)) * ones16
    w2_ref[...] = (z / (1.0 + z)) * ones16

    ind = oh1.astype(jnp.float32) + oh2.astype(jnp.float32)

    sub = lax.broadcasted_iota(jnp.int32, (SC, SC), 0)
    slt = (sub > lax.broadcasted_iota(jnp.int32, (SC, SC), 1)).astype(
        jnp.float32)

    nblk = S // SC
    tot = jnp.zeros((1, E), jnp.float32)
    r_blocks = []
    for k in range(nblk):
        ib = ind[k * SC:(k + 1) * SC, :]
        rb = lax.dot_general(
            slt, ib, (((1,), (0,)), ((), ())),
            preferred_element_type=jnp.float32,
        ) + tot
        r_blocks.append(rb)
        tot = tot + jnp.sum(ib, axis=0, keepdims=True)
    r = jnp.concatenate(r_blocks, axis=0)

    cnt = tot
    pc = jnp.ceil(cnt / TR) * TR
    e_i = lax.broadcasted_iota(jnp.int32, (E, E), 0)
    e_j = lax.broadcasted_iota(jnp.int32, (E, E), 1)
    elt = (e_i < e_j).astype(jnp.float32)
    off = lax.dot_general(
        pc, elt, (((1,), (0,)), ((), ())),
        preferred_element_type=jnp.float32,
    )

    dest = off + r
    pos1 = jnp.sum(jnp.where(oh1, dest, 0.0), axis=1, keepdims=True)
    pos2 = jnp.sum(jnp.where(oh2, dest, 0.0), axis=1, keepdims=True)
    pos1_ref[...] = pos1.astype(jnp.int32)
    pos2_ref[...] = pos2.astype(jnp.int32)

    used_f = jnp.sum(pc) / TR
    used_ref[...] = used_f.astype(jnp.int32).reshape(1, 1)

    offt = off / TR
    jio = lax.broadcasted_iota(jnp.int32, (NT, 1), 0).astype(jnp.float32)
    te = jnp.sum((offt <= jio).astype(jnp.int32), axis=1, keepdims=True) - 1
    te_ref[...] = te


@jax.jit
def _router(x2d, wgt, bg2d):
    return pl.pallas_call(
        _router_body,
        grid=(1,),
        in_specs=[
            pl.BlockSpec((S, D), lambda i: (0, 0)),
            pl.BlockSpec((D, E), lambda i: (0, 0)),
            pl.BlockSpec((1, E), lambda i: (0, 0)),
        ],
        out_specs=[
            pl.BlockSpec((S, WB), lambda i: (0, 0)),
            pl.BlockSpec((S, WB), lambda i: (0, 0)),
            pl.BlockSpec((S, 1), lambda i: (0, 0)),
            pl.BlockSpec((S, 1), lambda i: (0, 0)),
            pl.BlockSpec((NT, 1), lambda i: (0, 0)),
            pl.BlockSpec((1, 1), lambda i: (0, 0)),
        ],
        out_shape=[
            jax.ShapeDtypeStruct((S, WB), jnp.float32),
            jax.ShapeDtypeStruct((S, WB), jnp.float32),
            jax.ShapeDtypeStruct((S, 1), jnp.int32),
            jax.ShapeDtypeStruct((S, 1), jnp.int32),
            jax.ShapeDtypeStruct((NT, 1), jnp.int32),
            jax.ShapeDtypeStruct((1, 1), jnp.int32),
        ],
    )(x2d, wgt, bg2d)


def _gmm_body(te_ref, used_ref, xs_ref, ws_ref, we_ref, be_ref, y_ref):
    j = pl.program_id(0)

    @pl.when(j < used_ref[0])
    def _():
        prod = lax.dot_general(
            xs_ref[...], we_ref[0], (((1,), (1,)), ((), ())),
            preferred_element_type=jnp.float32,
        )
        y_ref[...] = (prod + be_ref[0]) * ws_ref[:, 0:1]


@jax.jit
def _gmm(te, used, xs, ws, we, be3):
    n_o = D // TO
    return pl.pallas_call(
        _gmm_body,
        grid_spec=pltpu.PrefetchScalarGridSpec(
            num_scalar_prefetch=2,
            grid=(NT, n_o),
            in_specs=[
                pl.BlockSpec((TR, D), lambda j, o, te, used: (j, 0)),
                pl.BlockSpec((TR, WB), lambda j, o, te, used: (j, 0)),
                pl.BlockSpec((1, TO, D), lambda j, o, te, used: (te[j], o, 0)),
                pl.BlockSpec((1, 1, TO), lambda j, o, te, used: (te[j], 0, o)),
            ],
            out_specs=pl.BlockSpec((TR, TO), lambda j, o, te, used: (j, o)),
        ),
        out_shape=jax.ShapeDtypeStruct((P, D), jnp.float32),
    )(te, used, xs, ws, we, be3)


NC = 2
NS = 16
NW = NC * NS
TOK_W = S // NW
CH = 16

_mesh = plsc.VectorSubcoreMesh(core_axis_name="c", subcore_axis_name="s")


@functools.partial(
    pl.kernel, mesh=_mesh,
    out_type=(
        jax.ShapeDtypeStruct((P, D), jnp.float32),
        jax.ShapeDtypeStruct((P, WB), jnp.float32),
    ),
    scratch_types=[
        pltpu.VMEM((CH,), jnp.int32),
        pltpu.VMEM((CH,), jnp.int32),
        pltpu.VMEM((CH, D), jnp.float32),
        pltpu.VMEM((CH, WB), jnp.float32),
        pltpu.VMEM((CH, WB), jnp.float32),
        pltpu.SemaphoreType.DMA,
        pltpu.SemaphoreType.DMA,
    ],
)
def _dispatch(x_hbm, pos1_hbm, pos2_hbm, w1b_hbm, w2b_hbm, xs_hbm, ws_hbm,
              idx1_v, idx2_v, rows_v, wr1_v, wr2_v, sem, wsem):
    wid = lax.axis_index("s") * NC + lax.axis_index("c")
    base = wid * TOK_W
    for c in range(TOK_W // CH):
        b = base + c * CH
        pltpu.sync_copy(pos1_hbm.at[pl.ds(b, CH)], idx1_v)
        pltpu.sync_copy(pos2_hbm.at[pl.ds(b, CH)], idx2_v)
        pltpu.sync_copy(x_hbm.at[pl.ds(b, CH)], rows_v)
        pltpu.sync_copy(w1b_hbm.at[pl.ds(b, CH)], wr1_v)
        pltpu.sync_copy(w2b_hbm.at[pl.ds(b, CH)], wr2_v)
        cpa = pltpu.async_copy(rows_v, xs_hbm.at[idx1_v], sem)
        cpb = pltpu.async_copy(rows_v, xs_hbm.at[idx2_v], sem)
        cpc = pltpu.async_copy(wr1_v, ws_hbm.at[idx1_v], wsem)
        cpd = pltpu.async_copy(wr2_v, ws_hbm.at[idx2_v], wsem)
        cpa.wait()
        cpb.wait()
        cpc.wait()
        cpd.wait()


@functools.partial(
    pl.kernel, mesh=_mesh,
    out_type=jax.ShapeDtypeStruct((S, D), jnp.float32),
    scratch_types=[
        pltpu.VMEM((CH,), jnp.int32),
        pltpu.VMEM((CH,), jnp.int32),
        pltpu.VMEM((CH, D), jnp.float32),
        pltpu.VMEM((CH, D), jnp.float32),
        pltpu.VMEM((CH, D), jnp.float32),
        pltpu.SemaphoreType.DMA,
        pltpu.SemaphoreType.DMA,
    ],
)
def _combine(y_hbm, pos1_hbm, pos2_hbm, out_hbm,
             idx1_v, idx2_v, y1_v, y2_v, o_v, sem1, sem2):
    wid = lax.axis_index("s") * NC + lax.axis_index("c")
    base = wid * TOK_W
    for c in range(TOK_W // CH):
        b = base + c * CH
        pltpu.sync_copy(pos1_hbm.at[pl.ds(b, CH)], idx1_v)
        pltpu.sync_copy(pos2_hbm.at[pl.ds(b, CH)], idx2_v)
        cp1 = pltpu.async_copy(y_hbm.at[idx1_v], y1_v, sem1)
        cp2 = pltpu.async_copy(y_hbm.at[idx2_v], y2_v, sem2)
        cp1.wait()
        cp2.wait()
        for i in range(CH):
            def add_row(v, _):
                sl = pl.ds(v * 16, 16)
                o_v[i, sl] = y1_v[i, sl] + y2_v[i, sl]
                return 0

            lax.fori_loop(0, D // 16, add_row, 0)
        pltpu.sync_copy(o_v, out_hbm.at[pl.ds(b, CH)])


def kernel(x, Wg, bg, We, be):
    B, S_, D_ = x.shape
    x2d = x.reshape(S_, D_)
    w1b, w2b, pos1, pos2, te, used = _router(x2d, Wg.T, bg.reshape(1, E))
    p1 = pos1.reshape(S)
    p2 = pos2.reshape(S)
    xs, ws = _dispatch(x2d, p1, p2, w1b, w2b)
    y = _gmm(te.reshape(NT), used.reshape(1), xs, ws, We,
             be.reshape(E, 1, D))
    out = _combine(y, p1, p2)
    return out.reshape(B, S_, D_)


# speedup vs baseline: 1.2641x; 1.2641x over previous
import functools

import jax
import jax.numpy as jnp
from jax import lax
from jax.experimental import pallas as pl
from jax.experimental.pallas import tpu as pltpu
from jax.experimental.pallas import tpu_sc as plsc

S = 2048
D = 2048
E = 8
TR = 256
NT = 24
P = NT * TR
TO = 2048
SC = 128
WB = 128

NEG = -jnp.inf


def _router_body(x_ref, wgt_ref, bg_ref, w1_ref, w2_ref, pos1_ref, pos2_ref,
                 te_ref, used_ref):
    xf = x_ref[...]
    logits = lax.dot_general(
        xf, wgt_ref[...], (((1,), (0,)), ((), ())),
        preferred_element_type=jnp.float32,
    ) + bg_ref[...]
    lane = lax.broadcasted_iota(jnp.int32, (S, E), 1)
    m1 = jnp.max(logits, axis=1, keepdims=True)
    i1 = jnp.min(jnp.where(logits == m1, lane, E), axis=1, keepdims=True)
    oh1 = lane == i1
    masked = jnp.where(oh1, NEG, logits)
    m2 = jnp.max(masked, axis=1, keepdims=True)
    i2 = jnp.min(jnp.where(masked == m2, lane, E), axis=1, keepdims=True)
    oh2 = lane == i2
    z = jnp.exp(m2 - m1)
    ones16 = jnp.ones((1, WB), jnp.float32)
    w1_ref[...] = (1.0 / (1.0 + z)) * ones16
    w2_ref[...] = (z / (1.0 + z)) * ones16

    ind = oh1.astype(jnp.float32) + oh2.astype(jnp.float32)

    sub = lax.broadcasted_iota(jnp.int32, (SC, SC), 0)
    slt = (sub > lax.broadcasted_iota(jnp.int32, (SC, SC), 1)).astype(
        jnp.float32)

    nblk = S // SC
    tot = jnp.zeros((1, E), jnp.float32)
    r_blocks = []
    for k in range(nblk):
        ib = ind[k * SC:(k + 1) * SC, :]
        rb = lax.dot_general(
            slt, ib, (((1,), (0,)), ((), ())),
            preferred_element_type=jnp.float32,
        ) + tot
        r_blocks.append(rb)
        tot = tot + jnp.sum(ib, axis=0, keepdims=True)
    r = jnp.concatenate(r_blocks, axis=0)

    cnt = tot
    pc = jnp.ceil(cnt / TR) * TR
    e_i = lax.broadcasted_iota(jnp.int32, (E, E), 0)
    e_j = lax.broadcasted_iota(jnp.int32, (E, E), 1)
    elt = (e_i < e_j).astype(jnp.float32)
    off = lax.dot_general(
        pc, elt, (((1,), (0,)), ((), ())),
        preferred_element_type=jnp.float32,
    )

    dest = off + r
    pos1 = jnp.sum(jnp.where(oh1, dest, 0.0), axis=1, keepdims=True)
    pos2 = jnp.sum(jnp.where(oh2, dest, 0.0), axis=1, keepdims=True)
    pos1_ref[...] = pos1.astype(jnp.int32)
    pos2_ref[...] = pos2.astype(jnp.int32)

    used_f = jnp.sum(pc) / TR
    used_ref[...] = used_f.astype(jnp.int32).reshape(1, 1)

    offt = off / TR
    jio = lax.broadcasted_iota(jnp.int32, (NT, 1), 0).astype(jnp.float32)
    te = jnp.sum((offt <= jio).astype(jnp.int32), axis=1, keepdims=True) - 1
    te_ref[...] = te


@jax.jit
def _router(x2d, wgt, bg2d):
    return pl.pallas_call(
        _router_body,
        grid=(1,),
        in_specs=[
            pl.BlockSpec((S, D), lambda i: (0, 0)),
            pl.BlockSpec((D, E), lambda i: (0, 0)),
            pl.BlockSpec((1, E), lambda i: (0, 0)),
        ],
        out_specs=[
            pl.BlockSpec((S, WB), lambda i: (0, 0)),
            pl.BlockSpec((S, WB), lambda i: (0, 0)),
            pl.BlockSpec((S, 1), lambda i: (0, 0)),
            pl.BlockSpec((S, 1), lambda i: (0, 0)),
            pl.BlockSpec((NT, 1), lambda i: (0, 0)),
            pl.BlockSpec((1, 1), lambda i: (0, 0)),
        ],
        out_shape=[
            jax.ShapeDtypeStruct((S, WB), jnp.float32),
            jax.ShapeDtypeStruct((S, WB), jnp.float32),
            jax.ShapeDtypeStruct((S, 1), jnp.int32),
            jax.ShapeDtypeStruct((S, 1), jnp.int32),
            jax.ShapeDtypeStruct((NT, 1), jnp.int32),
            jax.ShapeDtypeStruct((1, 1), jnp.int32),
        ],
    )(x2d, wgt, bg2d)


def _gmm_body(te_ref, used_ref, xs_ref, we_ref, be_ref, y_ref):
    j = pl.program_id(0)

    @pl.when(j < used_ref[0])
    def _():
        prod = lax.dot_general(
            xs_ref[...], we_ref[0], (((1,), (1,)), ((), ())),
            preferred_element_type=jnp.float32,
        )
        y_ref[...] = prod + be_ref[0]


@jax.jit
def _gmm(te, used, xs, we, be3):
    return pl.pallas_call(
        _gmm_body,
        grid_spec=pltpu.PrefetchScalarGridSpec(
            num_scalar_prefetch=2,
            grid=(NT,),
            in_specs=[
                pl.BlockSpec((TR, D), lambda j, te, used: (j, 0)),
                pl.BlockSpec((1, TO, D), lambda j, te, used: (te[j], 0, 0)),
                pl.BlockSpec((1, 1, TO), lambda j, te, used: (te[j], 0, 0)),
            ],
            out_specs=pl.BlockSpec((TR, TO), lambda j, te, used: (j, 0)),
        ),
        out_shape=jax.ShapeDtypeStruct((P, D), jnp.float32),
    )(te, used, xs, we, be3)


NC = 2
NS = 16
NW = NC * NS
TOK_W = S // NW
CH = 16

DCH = 16


@functools.cache
def _make_dispatch():
    mesh = plsc.VectorSubcoreMesh(core_axis_name="c", subcore_axis_name="s")
    return functools.partial(
        pl.kernel, mesh=mesh,
    out_type=jax.ShapeDtypeStruct((P, D), jnp.float32),
    scratch_types=[
        pltpu.VMEM((DCH,), jnp.int32),
        pltpu.VMEM((DCH,), jnp.int32),
        pltpu.VMEM((DCH, D), jnp.float32),
        pltpu.VMEM((DCH, D), jnp.float32),
        pltpu.SemaphoreType.DMA,
        pltpu.SemaphoreType.DMA,
        ],
    )(_dispatch_body)


def _dispatch_body(x_hbm, pos1_hbm, pos2_hbm, xs_hbm,
              idx1_v, idx2_v, rows0_v, rows1_v, sem, rsem):
    wid = lax.axis_index("s") * NC + lax.axis_index("c")
    base = wid * TOK_W
    nch = TOK_W // DCH
    rows = (rows0_v, rows1_v)
    pending = ()
    rh = pltpu.async_copy(x_hbm.at[pl.ds(base, DCH)], rows0_v, rsem)
    for c in range(nch):
        b = base + c * DCH
        for h in pending:
            h.wait()
        pltpu.sync_copy(pos1_hbm.at[pl.ds(b, DCH)], idx1_v)
        pltpu.sync_copy(pos2_hbm.at[pl.ds(b, DCH)], idx2_v)
        if c + 1 < nch:
            nrh = pltpu.async_copy(
                x_hbm.at[pl.ds(b + DCH, DCH)], rows[(c + 1) % 2], rsem)
        rh.wait()
        if c + 1 < nch:
            rh = nrh
        pending = (
            pltpu.async_copy(rows[c % 2], xs_hbm.at[idx1_v], sem),
            pltpu.async_copy(rows[c % 2], xs_hbm.at[idx2_v], sem),
        )
    for h in pending:
        h.wait()


CCH = 8


@functools.cache
def _make_combine():
    mesh = plsc.VectorSubcoreMesh(core_axis_name="c", subcore_axis_name="s")
    return functools.partial(
        pl.kernel, mesh=mesh,
    out_type=jax.ShapeDtypeStruct((S, D), jnp.float32),
    scratch_types=[
        pltpu.VMEM((CCH,), jnp.int32),
        pltpu.VMEM((CCH,), jnp.int32),
        pltpu.VMEM((CCH, WB), jnp.float32),
        pltpu.VMEM((CCH, WB), jnp.float32),
        pltpu.VMEM((CCH, D), jnp.float32),
        pltpu.VMEM((CCH, D), jnp.float32),
        pltpu.VMEM((CCH, D), jnp.float32),
        pltpu.VMEM((CCH, D), jnp.float32),
        pltpu.VMEM((CCH, D), jnp.float32),
        pltpu.VMEM((CCH, D), jnp.float32),
        pltpu.SemaphoreType.DMA,
        pltpu.SemaphoreType.DMA,
        pltpu.SemaphoreType.DMA,
        ],
    )(_combine_body)


def _combine_body(y_hbm, pos1_hbm, pos2_hbm, w1b_hbm, w2b_hbm, out_hbm,
             idx1_v, idx2_v, wr1_v, wr2_v,
             y1a_v, y2a_v, y1b_v, y2b_v, oa_v, ob_v, gsem, gsem2, osem):
    wid = lax.axis_index("s") * NC + lax.axis_index("c")
    base = wid * TOK_W
    nch = TOK_W // CCH
    y1 = (y1a_v, y1b_v)
    y2 = (y2a_v, y2b_v)
    ov = (oa_v, ob_v)

    def start_gathers(c):
        b = base + c * CCH
        pltpu.sync_copy(pos1_hbm.at[pl.ds(b, CCH)], idx1_v)
        pltpu.sync_copy(pos2_hbm.at[pl.ds(b, CCH)], idx2_v)
        return (
            pltpu.async_copy(y_hbm.at[idx1_v], y1[c % 2], gsem),
            pltpu.async_copy(y_hbm.at[idx2_v], y2[c % 2], gsem2),
        )

    gh = start_gathers(0)
    oh = ()
    for c in range(nch):
        b = base + c * CCH
        pltpu.sync_copy(w1b_hbm.at[pl.ds(b, CCH)], wr1_v)
        pltpu.sync_copy(w2b_hbm.at[pl.ds(b, CCH)], wr2_v)
        for h in gh:
            h.wait()
        if c + 1 < nch:
            ngh = start_gathers(c + 1)
        for h in oh:
            h.wait()
        y1c = y1[c % 2]
        y2c = y2[c % 2]
        oc = ov[c % 2]
        for i in range(CCH):
            wa = wr1_v[i, 0:16]
            wb = wr2_v[i, 0:16]

            def fma(v, _):
                s0 = pl.ds(v * 64, 16)
                s1 = pl.ds(v * 64 + 16, 16)
                s2 = pl.ds(v * 64 + 32, 16)
                s3 = pl.ds(v * 64 + 48, 16)
                oc[i, s0] = wa * y1c[i, s0] + wb * y2c[i, s0]
                oc[i, s1] = wa * y1c[i, s1] + wb * y2c[i, s1]
                oc[i, s2] = wa * y1c[i, s2] + wb * y2c[i, s2]
                oc[i, s3] = wa * y1c[i, s3] + wb * y2c[i, s3]
                return 0

            lax.fori_loop(0, D // 64, fma, 0)
        oh = (pltpu.async_copy(oc, out_hbm.at[pl.ds(b, CCH)], osem),)
        if c + 1 < nch:
            gh = ngh
    for h in oh:
        h.wait()


def kernel(x, Wg, bg, We, be):
    B, S_, D_ = x.shape
    x2d = x.reshape(S_, D_)
    w1b, w2b, pos1, pos2, te, used = _router(x2d, Wg.T, bg.reshape(1, E))
    p1 = pos1.reshape(S)
    p2 = pos2.reshape(S)
    xs = _make_dispatch()(x2d, p1, p2)
    y = _gmm(te.reshape(NT), used.reshape(1), xs, We, be.reshape(E, 1, D))
    out = _make_combine()(y, p1, p2, w1b, w2b)
    return out.reshape(B, S_, D_)


# speedup vs baseline: 1.3483x; 1.0667x over previous
import functools

import jax
import jax.numpy as jnp
from jax import lax
from jax.experimental import pallas as pl
from jax.experimental.pallas import tpu as pltpu
from jax.experimental.pallas import tpu_sc as plsc

S = 2048
D = 2048
E = 8
TR = 256
NT = 24
P = NT * TR
TO = 2048
SC = 128
WB = 128

NEG = -jnp.inf


def _router_body(x_ref, wgt_ref, bg_ref, w1_ref, w2_ref, pos1_ref, pos2_ref,
                 te_ref, used_ref):
    xf = x_ref[...]
    logits = lax.dot_general(
        xf, wgt_ref[...], (((1,), (0,)), ((), ())),
        preferred_element_type=jnp.float32,
    ) + bg_ref[...]
    lane = lax.broadcasted_iota(jnp.int32, (S, E), 1)
    m1 = jnp.max(logits, axis=1, keepdims=True)
    i1 = jnp.min(jnp.where(logits == m1, lane, E), axis=1, keepdims=True)
    oh1 = lane == i1
    masked = jnp.where(oh1, NEG, logits)
    m2 = jnp.max(masked, axis=1, keepdims=True)
    i2 = jnp.min(jnp.where(masked == m2, lane, E), axis=1, keepdims=True)
    oh2 = lane == i2
    z = jnp.exp(m2 - m1)
    ones16 = jnp.ones((1, WB), jnp.float32)
    w1_ref[...] = (1.0 / (1.0 + z)) * ones16
    w2_ref[...] = (z / (1.0 + z)) * ones16

    ind = oh1.astype(jnp.float32) + oh2.astype(jnp.float32)

    sub = lax.broadcasted_iota(jnp.int32, (SC, SC), 0)
    slt = (sub > lax.broadcasted_iota(jnp.int32, (SC, SC), 1)).astype(
        jnp.float32)

    nblk = S // SC
    tot = jnp.zeros((1, E), jnp.float32)
    r_blocks = []
    for k in range(nblk):
        ib = ind[k * SC:(k + 1) * SC, :]
        rb = lax.dot_general(
            slt, ib, (((1,), (0,)), ((), ())),
            preferred_element_type=jnp.float32,
        ) + tot
        r_blocks.append(rb)
        tot = tot + jnp.sum(ib, axis=0, keepdims=True)
    r = jnp.concatenate(r_blocks, axis=0)

    cnt = tot
    pc = jnp.ceil(cnt / TR) * TR
    e_i = lax.broadcasted_iota(jnp.int32, (E, E), 0)
    e_j = lax.broadcasted_iota(jnp.int32, (E, E), 1)
    elt = (e_i < e_j).astype(jnp.float32)
    off = lax.dot_general(
        pc, elt, (((1,), (0,)), ((), ())),
        preferred_element_type=jnp.float32,
    )

    dest = off + r
    pos1 = jnp.sum(jnp.where(oh1, dest, 0.0), axis=1, keepdims=True)
    pos2 = jnp.sum(jnp.where(oh2, dest, 0.0), axis=1, keepdims=True)
    pos1_ref[...] = pos1.astype(jnp.int32)
    pos2_ref[...] = pos2.astype(jnp.int32)

    used_f = jnp.sum(pc) / TR
    used_ref[...] = used_f.astype(jnp.int32).reshape(1, 1)

    offt = off / TR
    jio = lax.broadcasted_iota(jnp.int32, (NT, 1), 0).astype(jnp.float32)
    te = jnp.sum((offt <= jio).astype(jnp.int32), axis=1, keepdims=True) - 1
    te_ref[...] = te


@jax.jit
def _router(x2d, wgt, bg2d):
    return pl.pallas_call(
        _router_body,
        grid=(1,),
        in_specs=[
            pl.BlockSpec((S, D), lambda i: (0, 0)),
            pl.BlockSpec((D, E), lambda i: (0, 0)),
            pl.BlockSpec((1, E), lambda i: (0, 0)),
        ],
        out_specs=[
            pl.BlockSpec((S, WB), lambda i: (0, 0)),
            pl.BlockSpec((S, WB), lambda i: (0, 0)),
            pl.BlockSpec((S, 1), lambda i: (0, 0)),
            pl.BlockSpec((S, 1), lambda i: (0, 0)),
            pl.BlockSpec((NT, 1), lambda i: (0, 0)),
            pl.BlockSpec((1, 1), lambda i: (0, 0)),
        ],
        out_shape=[
            jax.ShapeDtypeStruct((S, WB), jnp.float32),
            jax.ShapeDtypeStruct((S, WB), jnp.float32),
            jax.ShapeDtypeStruct((S, 1), jnp.int32),
            jax.ShapeDtypeStruct((S, 1), jnp.int32),
            jax.ShapeDtypeStruct((NT, 1), jnp.int32),
            jax.ShapeDtypeStruct((1, 1), jnp.int32),
        ],
    )(x2d, wgt, bg2d)


def _gmm_body(te_ref, used_ref, xs_ref, we_ref, be_ref, y_ref):
    j = pl.program_id(0)

    @pl.when(j < used_ref[0])
    def _():
        prod = lax.dot_general(
            xs_ref[...], we_ref[0], (((1,), (1,)), ((), ())),
            preferred_element_type=jnp.float32,
        )
        y_ref[...] = prod + be_ref[0]


@jax.jit
def _gmm(te, used, xs, we, be3):
    return pl.pallas_call(
        _gmm_body,
        grid_spec=pltpu.PrefetchScalarGridSpec(
            num_scalar_prefetch=2,
            grid=(NT,),
            in_specs=[
                pl.BlockSpec((TR, D), lambda j, te, used: (j, 0)),
                pl.BlockSpec((1, TO, D), lambda j, te, used: (te[j], 0, 0)),
                pl.BlockSpec((1, 1, TO), lambda j, te, used: (te[j], 0, 0)),
            ],
            out_specs=pl.BlockSpec((TR, TO), lambda j, te, used: (j, 0)),
        ),
        out_shape=jax.ShapeDtypeStruct((P, D), jnp.float32),
    )(te, used, xs, we, be3)


NC = 2
NS = 16
NW = NC * NS
TOK_W = S // NW
CH = 16

DCH = 16


@functools.cache
def _make_dispatch():
    mesh = plsc.VectorSubcoreMesh(core_axis_name="c", subcore_axis_name="s")
    return functools.partial(
        pl.kernel, mesh=mesh,
    out_type=jax.ShapeDtypeStruct((P, D), jnp.float32),
    scratch_types=[
        pltpu.VMEM((2, TOK_W // DCH, DCH), jnp.int32),
        pltpu.VMEM((DCH, D), jnp.float32),
        pltpu.VMEM((DCH, D), jnp.float32),
        pltpu.SemaphoreType.DMA,
        pltpu.SemaphoreType.DMA,
        pltpu.SemaphoreType.DMA,
        ],
    )(_dispatch_body)


def _dispatch_body(x_hbm, pos1_hbm, pos2_hbm, xs_hbm,
                   idx_v, rows0_v, rows1_v, sem, rsem, isem):
    wid = lax.axis_index("s") * NC + lax.axis_index("c")
    base = wid * TOK_W
    nch = TOK_W // DCH
    rbase = wid * nch
    ih1 = pltpu.async_copy(pos1_hbm.at[pl.ds(rbase, nch)], idx_v.at[0], isem)
    ih2 = pltpu.async_copy(pos2_hbm.at[pl.ds(rbase, nch)], idx_v.at[1], isem)
    rows = (rows0_v, rows1_v)
    pending = ()
    rh = pltpu.async_copy(x_hbm.at[pl.ds(base, DCH)], rows0_v, rsem)
    ih1.wait()
    ih2.wait()
    for c in range(nch):
        b = base + c * DCH
        for h in pending:
            h.wait()
        if c + 1 < nch:
            nrh = pltpu.async_copy(
                x_hbm.at[pl.ds(b + DCH, DCH)], rows[(c + 1) % 2], rsem)
        rh.wait()
        if c + 1 < nch:
            rh = nrh
        pending = (
            pltpu.async_copy(rows[c % 2], xs_hbm.at[idx_v.at[0, c]], sem),
            pltpu.async_copy(rows[c % 2], xs_hbm.at[idx_v.at[1, c]], sem),
        )
    for h in pending:
        h.wait()


CCH = 8


@functools.cache
def _make_combine():
    mesh = plsc.VectorSubcoreMesh(core_axis_name="c", subcore_axis_name="s")
    return functools.partial(
        pl.kernel, mesh=mesh,
    out_type=jax.ShapeDtypeStruct((S, D), jnp.float32),
    scratch_types=[
        pltpu.VMEM((2, TOK_W // CCH, CCH), jnp.int32),
        pltpu.VMEM((TOK_W, WB), jnp.float32),
        pltpu.VMEM((TOK_W, WB), jnp.float32),
        pltpu.VMEM((CCH, D), jnp.float32),
        pltpu.VMEM((CCH, D), jnp.float32),
        pltpu.VMEM((CCH, D), jnp.float32),
        pltpu.VMEM((CCH, D), jnp.float32),
        pltpu.VMEM((CCH, D), jnp.float32),
        pltpu.VMEM((CCH, D), jnp.float32),
        pltpu.SemaphoreType.DMA,
        pltpu.SemaphoreType.DMA,
        pltpu.SemaphoreType.DMA,
        pltpu.SemaphoreType.DMA,
        ],
    )(_combine_body)


def _combine_body(y_hbm, pos1_hbm, pos2_hbm, w1b_hbm, w2b_hbm, out_hbm,
                  idx_v, w1a_v, w2a_v,
                  y1a_v, y2a_v, y1b_v, y2b_v, oa_v, ob_v, gsem, gsem2, osem,
                  isem):
    wid = lax.axis_index("s") * NC + lax.axis_index("c")
    base = wid * TOK_W
    nch = TOK_W // CCH
    rbase = wid * nch
    ih1 = pltpu.async_copy(pos1_hbm.at[pl.ds(rbase, nch)], idx_v.at[0], isem)
    ih2 = pltpu.async_copy(pos2_hbm.at[pl.ds(rbase, nch)], idx_v.at[1], isem)
    wh1 = pltpu.async_copy(w1b_hbm.at[pl.ds(base, TOK_W)], w1a_v, isem)
    wh2 = pltpu.async_copy(w2b_hbm.at[pl.ds(base, TOK_W)], w2a_v, isem)
    y1 = (y1a_v, y1b_v)
    y2 = (y2a_v, y2b_v)
    ov = (oa_v, ob_v)
    ih1.wait()
    ih2.wait()

    def start_gathers(c):
        return (
            pltpu.async_copy(y_hbm.at[idx_v.at[0, c]], y1[c % 2], gsem),
            pltpu.async_copy(y_hbm.at[idx_v.at[1, c]], y2[c % 2], gsem2),
        )

    gh = start_gathers(0)
    wh1.wait()
    wh2.wait()
    oh = ()
    for c in range(nch):
        b = base + c * CCH
        for h in gh:
            h.wait()
        if c + 1 < nch:
            ngh = start_gathers(c + 1)
        for h in oh:
            h.wait()
        y1c = y1[c % 2]
        y2c = y2[c % 2]
        oc = ov[c % 2]
        for i in range(CCH):
            t = c * CCH + i
            wa = w1a_v[t, 0:16]
            wb = w2a_v[t, 0:16]

            def fma(v, _):
                s0 = pl.ds(v * 64, 16)
                s1 = pl.ds(v * 64 + 16, 16)
                s2 = pl.ds(v * 64 + 32, 16)
                s3 = pl.ds(v * 64 + 48, 16)
                oc[i, s0] = wa * y1c[i, s0] + wb * y2c[i, s0]
                oc[i, s1] = wa * y1c[i, s1] + wb * y2c[i, s1]
                oc[i, s2] = wa * y1c[i, s2] + wb * y2c[i, s2]
                oc[i, s3] = wa * y1c[i, s3] + wb * y2c[i, s3]
                return 0

            lax.fori_loop(0, D // 64, fma, 0)
        oh = (pltpu.async_copy(oc, out_hbm.at[pl.ds(b, CCH)], osem),)
        if c + 1 < nch:
            gh = ngh
    for h in oh:
        h.wait()


def kernel(x, Wg, bg, We, be):
    B, S_, D_ = x.shape
    x2d = x.reshape(S_, D_)
    w1b, w2b, pos1, pos2, te, used = _router(x2d, Wg.T, bg.reshape(1, E))
    p1d = pos1.reshape(S // DCH, DCH)
    p2d = pos2.reshape(S // DCH, DCH)
    p1c = pos1.reshape(S // CCH, CCH)
    p2c = pos2.reshape(S // CCH, CCH)
    xs = _make_dispatch()(x2d, p1d, p2d)
    y = _gmm(te.reshape(NT), used.reshape(1), xs, We, be.reshape(E, 1, D))
    out = _make_combine()(y, p1c, p2c, w1b, w2b)
    return out.reshape(B, S_, D_)
